# baseline (speedup 1.0000x reference)
import functools

import jax
import jax.numpy as jnp
from jax import lax
from jax.experimental import pallas as pl
from jax.experimental.pallas import tpu as pltpu
from jax.experimental.pallas import tpu_sc as plsc

N_CAT = 26
N_NUM = 13
VOCAB = 100000
DIM = 32
BATCH = 16384
EPS = 1e-5
EMB_F = N_CAT * DIM
FEAT = EMB_F + N_NUM

NW = 32
ROWS_W = EMB_F // NW
CH_B = 4096
NCH = BATCH // CH_B
IDXR = CH_B // 128


def _gather_sc(tabT, idx2):
    mesh = plsc.VectorSubcoreMesh(core_axis_name="c", subcore_axis_name="s")

    @functools.partial(
        pl.kernel,
        mesh=mesh,
        out_type=jax.ShapeDtypeStruct((EMB_F, BATCH), jnp.float32),
        compiler_params=pltpu.CompilerParams(
            use_tc_tiling_on_sc=True, needs_layout_passes=False),
        scratch_types=[
            pltpu.VMEM((VOCAB,), jnp.float32),
            pltpu.VMEM((BATCH // 128, 128), jnp.int32),
            pltpu.VMEM((2, CH_B), jnp.float32),
            pltpu.SemaphoreType.DMA,
        ],
    )
    def gk(tab, idx, out, row_v, idxc_v, out_v, sem):
        wid = lax.axis_index("s") * 2 + lax.axis_index("c")

        def row_body(k, iprev):
            f = wid * ROWS_W + k
            i = f // DIM
            pltpu.sync_copy(tab.at[f], row_v)

            @pl.when(i != iprev)
            def _():
                pltpu.sync_copy(idx.at[pl.ds(i * 128, 128)], idxc_v)

            def chunk_body(c, carry2):
                b = c % 2
                g = k * NCH + c

                @pl.when(g >= 2)
                def _():
                    pltpu.make_async_copy(
                        out_v.at[0], out.at[0, pl.ds(0, CH_B)], sem).wait()

                def j_body(j, carry3):
                    vals = []
                    for u in range(16):
                        jj = j * 16 + u
                        iv = idxc_v[c * IDXR + jj // 8, pl.ds((jj % 8) * 16, 16)]
                        vals.append(plsc.load_gather(row_v, [iv]))
                    for u in range(16):
                        out_v[b, pl.ds((j * 16 + u) * 16, 16)] = vals[u]
                    return carry3

                lax.fori_loop(0, CH_B // 256, j_body, 0)

                pltpu.make_async_copy(
                    out_v.at[b], out.at[f, pl.ds(c * CH_B, CH_B)], sem).start()
                return carry2

            lax.fori_loop(0, NCH, chunk_body, 0)
            return i

        lax.fori_loop(0, ROWS_W, row_body, -1)
        for _ in range(2):
            pltpu.make_async_copy(
                out_v.at[0], out.at[0, pl.ds(0, CH_B)], sem).wait()

    return gk(tabT, idx2)


BSL = 2048
NBL = BATCH // BSL


def _stats_tc(embT, x_numT, ge, gn, be, bn):
    def stats_kernel(emb_ref, num_ref, ge_ref, gn_ref, be_ref, bn_ref,
                     se_ref, sn_ref, he_ref, hn_ref, s1, s2, n1, n2):
        j = pl.program_id(0)
        e = emb_ref[...]
        x = num_ref[...]
        pe = jnp.sum(e, axis=1, keepdims=True)
        pe2 = jnp.sum(e * e, axis=1, keepdims=True)
        pn = jnp.sum(x, axis=1, keepdims=True)
        pn2 = jnp.sum(x * x, axis=1, keepdims=True)

        @pl.when(j == 0)
        def _():
            s1[...] = pe
            s2[...] = pe2
            n1[...] = pn
            n2[...] = pn2

        @pl.when(j > 0)
        def _():
            s1[...] += pe
            s2[...] += pe2
            n1[...] += pn
            n2[...] += pn2

        @pl.when(j == NBL - 1)
        def _():
            inv_b = jnp.float32(1.0 / BATCH)
            me = s1[...] * inv_b
            ve = s2[...] * inv_b - me * me
            re = lax.rsqrt(ve + EPS)
            mn = n1[...] * inv_b
            vn = n2[...] * inv_b - mn * mn
            rn = lax.rsqrt(vn + EPS)
            sc_e = ge_ref[...] * re
            sc_n = gn_ref[...] * rn
            se_ref[...] = sc_e
            sn_ref[...] = sc_n
            he_ref[...] = be_ref[...] - me * sc_e
            hn_ref[...] = bn_ref[...] - mn * sc_n

    return pl.pallas_call(
        stats_kernel,
        grid=(NBL,),
        in_specs=[
            pl.BlockSpec((EMB_F, BSL), lambda j: (0, j)),
            pl.BlockSpec((N_NUM, BSL), lambda j: (0, j)),
            pl.BlockSpec((EMB_F, 1), lambda j: (0, 0)),
            pl.BlockSpec((N_NUM, 1), lambda j: (0, 0)),
            pl.BlockSpec((EMB_F, 1), lambda j: (0, 0)),
            pl.BlockSpec((N_NUM, 1), lambda j: (0, 0)),
        ],
        out_specs=[
            pl.BlockSpec((EMB_F, 1), lambda j: (0, 0)),
            pl.BlockSpec((N_NUM, 1), lambda j: (0, 0)),
            pl.BlockSpec((EMB_F, 1), lambda j: (0, 0)),
            pl.BlockSpec((N_NUM, 1), lambda j: (0, 0)),
        ],
        out_shape=[
            jax.ShapeDtypeStruct((EMB_F, 1), jnp.float32),
            jax.ShapeDtypeStruct((N_NUM, 1), jnp.float32),
            jax.ShapeDtypeStruct((EMB_F, 1), jnp.float32),
            jax.ShapeDtypeStruct((N_NUM, 1), jnp.float32),
        ],
        scratch_shapes=[
            pltpu.VMEM((EMB_F, 1), jnp.float32),
            pltpu.VMEM((EMB_F, 1), jnp.float32),
            pltpu.VMEM((N_NUM, 1), jnp.float32),
            pltpu.VMEM((N_NUM, 1), jnp.float32),
        ],
    )(embT, x_numT, ge, gn, be, bn)


def _norm_tc(embT, x_numT, se, sn, he, hn):
    def norm_kernel(emb_ref, num_ref, se_ref, sn_ref, he_ref, hn_ref, out_ref):
        e = emb_ref[...] * se_ref[...] + he_ref[...]
        x = num_ref[...] * sn_ref[...] + hn_ref[...]
        out_ref[...] = jnp.concatenate([e, x], axis=0)

    return pl.pallas_call(
        norm_kernel,
        grid=(NBL,),
        in_specs=[
            pl.BlockSpec((EMB_F, BSL), lambda j: (0, j)),
            pl.BlockSpec((N_NUM, BSL), lambda j: (0, j)),
            pl.BlockSpec((EMB_F, 1), lambda j: (0, 0)),
            pl.BlockSpec((N_NUM, 1), lambda j: (0, 0)),
            pl.BlockSpec((EMB_F, 1), lambda j: (0, 0)),
            pl.BlockSpec((N_NUM, 1), lambda j: (0, 0)),
        ],
        out_specs=pl.BlockSpec((FEAT, BSL), lambda j: (0, j)),
        out_shape=jax.ShapeDtypeStruct((FEAT, BATCH), jnp.float32),
    )(embT, x_numT, se, sn, he, hn)


def kernel(x_num, x_cat, tables, gamma, beta):
    x_cat = x_cat.astype(jnp.int32)
    tabT = tables.transpose(0, 2, 1).reshape(EMB_F, VOCAB)
    idx2 = x_cat.T.reshape(EMB_F * BATCH // (DIM * 128), 128)
    embT = _gather_sc(tabT, idx2)
    x_numT = x_num.T
    ge = gamma[:EMB_F].reshape(EMB_F, 1)
    gn = gamma[EMB_F:].reshape(N_NUM, 1)
    be = beta[:EMB_F].reshape(EMB_F, 1)
    bn = beta[EMB_F:].reshape(N_NUM, 1)
    se, sn, he, hn = _stats_tc(embT, x_numT, ge, gn, be, bn)
    outT = _norm_tc(embT, x_numT, se, sn, he, hn)
    return outT.T


# speedup vs baseline: 1.0654x; 1.0654x over previous
import functools

import jax
import jax.numpy as jnp
from jax import lax
from jax.experimental import pallas as pl
from jax.experimental.pallas import tpu as pltpu
from jax.experimental.pallas import tpu_sc as plsc

N_CAT = 26
N_NUM = 13
VOCAB = 100000
DIM = 32
BATCH = 16384
EPS = 1e-5
EMB_F = N_CAT * DIM
FEAT = EMB_F + N_NUM

NW = 32
ROWS_W = EMB_F // NW
CH_B = 4096
NCH = BATCH // CH_B
IDXR = CH_B // 128


def _gather_sc(tabT, idx2):
    mesh = plsc.VectorSubcoreMesh(core_axis_name="c", subcore_axis_name="s")

    @functools.partial(
        pl.kernel,
        mesh=mesh,
        out_type=(
            jax.ShapeDtypeStruct((EMB_F, BATCH), jnp.float32),
            jax.ShapeDtypeStruct((NW, ROWS_W * 32), jnp.float32),
        ),
        compiler_params=pltpu.CompilerParams(
            use_tc_tiling_on_sc=True, needs_layout_passes=False),
        scratch_types=[
            pltpu.VMEM((VOCAB,), jnp.float32),
            pltpu.VMEM((BATCH // 128, 128), jnp.int32),
            pltpu.VMEM((2, CH_B), jnp.float32),
            pltpu.VMEM((ROWS_W * 32,), jnp.float32),
            pltpu.SemaphoreType.DMA,
        ],
    )
    def gk(tab, idx, out, stats, row_v, idxc_v, out_v, stats_v, sem):
        wid = lax.axis_index("s") * 2 + lax.axis_index("c")

        def row_body(k, iprev):
            f = wid * ROWS_W + k
            i = f // DIM
            pltpu.sync_copy(tab.at[f], row_v)

            @pl.when(i != iprev)
            def _():
                pltpu.sync_copy(idx.at[pl.ds(i * 128, 128)], idxc_v)

            def chunk_body(c, acc):
                s1, s2 = acc
                b = c % 2
                g = k * NCH + c

                @pl.when(g >= 2)
                def _():
                    pltpu.make_async_copy(
                        out_v.at[0], out.at[0, pl.ds(0, CH_B)], sem).wait()

                def j_body(j, acc3):
                    s1, s2 = acc3
                    vals = []
                    for u in range(16):
                        jj = j * 16 + u
                        iv = idxc_v[c * IDXR + jj // 8, pl.ds((jj % 8) * 16, 16)]
                        vals.append(plsc.load_gather(row_v, [iv]))
                    for u in range(16):
                        out_v[b, pl.ds((j * 16 + u) * 16, 16)] = vals[u]
                        s1 = s1 + vals[u]
                        s2 = s2 + vals[u] * vals[u]
                    return (s1, s2)

                s1, s2 = lax.fori_loop(0, CH_B // 256, j_body, (s1, s2))
                pltpu.make_async_copy(
                    out_v.at[b], out.at[f, pl.ds(c * CH_B, CH_B)], sem).start()
                return (s1, s2)

            zero = jnp.zeros((16,), jnp.float32)
            s1, s2 = lax.fori_loop(0, NCH, chunk_body, (zero, zero))
            stats_v[pl.ds(k * 32, 16)] = s1
            stats_v[pl.ds(k * 32 + 16, 16)] = s2
            return i

        lax.fori_loop(0, ROWS_W, row_body, -1)
        pltpu.sync_copy(stats_v, stats.at[wid])
        for _ in range(2):
            pltpu.make_async_copy(
                out_v.at[0], out.at[0, pl.ds(0, CH_B)], sem).wait()

    return gk(tabT, idx2)


def _xstats_tc(x_numT, gn, bn):
    def xstats_kernel(num_ref, gn_ref, bn_ref, sn_ref, hn_ref):
        x = num_ref[...]
        inv_b = jnp.float32(1.0 / BATCH)
        mn = jnp.sum(x, axis=1, keepdims=True) * inv_b
        vn = jnp.sum(x * x, axis=1, keepdims=True) * inv_b - mn * mn
        rn = lax.rsqrt(vn + EPS)
        sc = gn_ref[...] * rn
        sn_ref[...] = sc
        hn_ref[...] = bn_ref[...] - mn * sc

    return pl.pallas_call(
        xstats_kernel,
        grid=(1,),
        in_specs=[
            pl.BlockSpec((N_NUM, BATCH), lambda j: (0, 0)),
            pl.BlockSpec((N_NUM, 1), lambda j: (0, 0)),
            pl.BlockSpec((N_NUM, 1), lambda j: (0, 0)),
        ],
        out_specs=[
            pl.BlockSpec((N_NUM, 1), lambda j: (0, 0)),
            pl.BlockSpec((N_NUM, 1), lambda j: (0, 0)),
        ],
        out_shape=[
            jax.ShapeDtypeStruct((N_NUM, 1), jnp.float32),
            jax.ShapeDtypeStruct((N_NUM, 1), jnp.float32),
        ],
    )(x_numT, gn, bn)


BSL = 2048
NBL = BATCH // BSL


def _norm_tc(embT, x_numT, se, sn, he, hn):
    def norm_kernel(emb_ref, num_ref, se_ref, sn_ref, he_ref, hn_ref, out_ref):
        e = emb_ref[...] * se_ref[...] + he_ref[...]
        x = num_ref[...] * sn_ref[...] + hn_ref[...]
        out_ref[...] = jnp.concatenate([e, x], axis=0)

    return pl.pallas_call(
        norm_kernel,
        grid=(NBL,),
        in_specs=[
            pl.BlockSpec((EMB_F, BSL), lambda j: (0, j)),
            pl.BlockSpec((N_NUM, BSL), lambda j: (0, j)),
            pl.BlockSpec((EMB_F, 1), lambda j: (0, 0)),
            pl.BlockSpec((N_NUM, 1), lambda j: (0, 0)),
            pl.BlockSpec((EMB_F, 1), lambda j: (0, 0)),
            pl.BlockSpec((N_NUM, 1), lambda j: (0, 0)),
        ],
        out_specs=pl.BlockSpec((FEAT, BSL), lambda j: (0, j)),
        out_shape=jax.ShapeDtypeStruct((FEAT, BATCH), jnp.float32),
    )(embT, x_numT, se, sn, he, hn)


def kernel(x_num, x_cat, tables, gamma, beta):
    x_cat = x_cat.astype(jnp.int32)
    tabT = tables.transpose(0, 2, 1).reshape(EMB_F, VOCAB)
    idx2 = x_cat.T.reshape(EMB_F * BATCH // (DIM * 128), 128)
    embT, stats = _gather_sc(tabT, idx2)
    x_numT = x_num.T
    ge = gamma[:EMB_F].reshape(EMB_F, 1)
    gn = gamma[EMB_F:].reshape(N_NUM, 1)
    be = beta[:EMB_F].reshape(EMB_F, 1)
    bn = beta[EMB_F:].reshape(N_NUM, 1)
    p = stats.reshape(NW, ROWS_W, 2, 16).sum(axis=3)
    s = p[:, :, 0].reshape(EMB_F, 1)
    ss = p[:, :, 1].reshape(EMB_F, 1)
    inv_b = jnp.float32(1.0 / BATCH)
    me = s * inv_b
    ve = ss * inv_b - me * me
    re = lax.rsqrt(ve + EPS)
    se = ge * re
    he = be - me * se
    sn, hn = _xstats_tc(x_numT, gn, bn)
    outT = _norm_tc(embT, x_numT, se, sn, he, hn)
    return outT.T
